# baseline (speedup 1.0000x reference)
import functools

import jax
import jax.numpy as jnp
from jax import lax
from jax.experimental import pallas as pl
from jax.experimental.pallas import tpu as pltpu
from jax.experimental.pallas import tpu_sc as plsc

E = 16; G = 4; EPG = 4; K = 2; TOPK_GROUP = 2
C = 2048; H = 1024; H_SHARED = 1024

BLK = 256
NEG_INF = float("-inf")


def _num_blocks(S):
    SK = S * K
    return -(-(SK + E * (BLK - 1)) // BLK)


def _router_body(x_ref, rw_ref, rb_ref, topk_ref, fw_ref):
    xb = x_ref[...]
    logits = jax.lax.dot_general(xb, rw_ref[...], (((1,), (1,)), ((), ())),
                                 preferred_element_type=jnp.float32)
    scores = jax.nn.sigmoid(logits)
    sb = scores + rb_ref[...]

    TS = xb.shape[0]
    le = lax.broadcasted_iota(jnp.int32, (TS, E), 1)
    gl = le // EPG

    gcols = []
    for g_id in range(G):
        gm = jnp.max(jnp.where(gl == g_id, sb, NEG_INF), axis=1, keepdims=True)
        gcols.append(gm)
    gsc = jnp.concatenate(gcols, axis=1)

    gi = lax.broadcasted_iota(jnp.int32, (TS, G), 1)
    m1 = jnp.max(gsc, axis=1, keepdims=True)
    g1 = jnp.min(jnp.where(gsc == m1, gi, G), axis=1, keepdims=True)
    gsc2 = jnp.where(gi == g1, NEG_INF, gsc)
    m2 = jnp.max(gsc2, axis=1, keepdims=True)
    g2 = jnp.min(jnp.where(gsc2 == m2, gi, G), axis=1, keepdims=True)

    gmask = (gl == g1) | (gl == g2)
    msb = jnp.where(gmask, sb, NEG_INF)

    v1 = jnp.max(msb, axis=1, keepdims=True)
    e1 = jnp.min(jnp.where(msb == v1, le, E), axis=1, keepdims=True)
    msb2 = jnp.where(le == e1, NEG_INF, msb)
    v2 = jnp.max(msb2, axis=1, keepdims=True)
    e2 = jnp.min(jnp.where(msb2 == v2, le, E), axis=1, keepdims=True)

    w1 = jnp.sum(jnp.where(le == e1, scores, 0.0), axis=1, keepdims=True)
    w2 = jnp.sum(jnp.where(le == e2, scores, 0.0), axis=1, keepdims=True)
    norm = w1 + w2 + 1e-20

    topk_ref[...] = jnp.concatenate([e1, e2], axis=1)
    fw_ref[...] = jnp.concatenate([w1 / norm, w2 / norm], axis=1)


def _router(xf, router_w, corr_bias):
    S = xf.shape[0]
    TS = 512
    return pl.pallas_call(
        _router_body,
        grid=(S // TS,),
        in_specs=[
            pl.BlockSpec((TS, C), lambda i: (i, 0)),
            pl.BlockSpec((E, C), lambda i: (0, 0)),
            pl.BlockSpec((1, E), lambda i: (0, 0)),
        ],
        out_specs=[
            pl.BlockSpec((TS, K), lambda i: (i, 0)),
            pl.BlockSpec((TS, K), lambda i: (i, 0)),
        ],
        out_shape=[
            jax.ShapeDtypeStruct((S, K), jnp.int32),
            jax.ShapeDtypeStruct((S, K), jnp.float32),
        ],
    )(xf, router_w, corr_bias.reshape(1, E))


def _shared_body(x_ref, sg_ref, su_ref, sd_ref, out_ref):
    xb = x_ref[...]
    g = jax.lax.dot_general(xb, sg_ref[...], (((1,), (1,)), ((), ())),
                            preferred_element_type=jnp.float32)
    u = jax.lax.dot_general(xb, su_ref[...], (((1,), (1,)), ((), ())),
                            preferred_element_type=jnp.float32)
    h = (g * jax.nn.sigmoid(g)) * u
    out_ref[...] = jax.lax.dot_general(h, sd_ref[...], (((1,), (1,)), ((), ())),
                                       preferred_element_type=jnp.float32)


def _shared_expert(xf, sgw, suw, sdw):
    S = xf.shape[0]
    TS = 256
    return pl.pallas_call(
        _shared_body,
        grid=(S // TS,),
        in_specs=[
            pl.BlockSpec((TS, C), lambda i: (i, 0)),
            pl.BlockSpec((H_SHARED, C), lambda i: (0, 0)),
            pl.BlockSpec((H_SHARED, C), lambda i: (0, 0)),
            pl.BlockSpec((C, H_SHARED), lambda i: (0, 0)),
        ],
        out_specs=pl.BlockSpec((TS, C), lambda i: (i, 0)),
        out_shape=jax.ShapeDtypeStruct((S, C), jnp.float32),
    )(xf, sgw, suw, sdw)


def _dispatch_metadata(topk, fw, S, NW, n_ch, CHT):
    SK = S * K
    NB = _num_blocks(S)
    P = NB * BLK

    e_flat = topk.reshape(SK)
    oh = e_flat[:, None] == jnp.arange(E, dtype=jnp.int32)[None, :]
    ohi = oh.astype(jnp.int32)
    cum = jnp.cumsum(ohi, axis=0)
    counts = cum[-1]
    rank = jnp.sum(jnp.where(oh, cum - 1, 0), axis=1)

    padded = -(-counts // BLK) * BLK
    pad_end = jnp.cumsum(padded).astype(jnp.int32)
    pad_start = pad_end - padded
    ppos = pad_start[e_flat] + rank

    bstart = jnp.arange(NB, dtype=jnp.int32) * BLK
    beid = jnp.searchsorted(pad_end, bstart, side="right").astype(jnp.int32)
    beidc = jnp.minimum(beid, E - 1)
    valid = ((beid < E) &
             ((bstart - pad_start[beidc]) < counts[beidc])).astype(jnp.int32)
    beid_f = lax.associative_scan(jnp.maximum, jnp.where(valid == 1, beidc, 0))

    psc = (ppos.reshape(NW, n_ch, CHT, K)
               .transpose(0, 1, 3, 2)
               .reshape(NW, K * n_ch, CHT))
    rpw = S // NW
    pos_pl = ppos.reshape(NW, rpw, K).transpose(0, 2, 1)
    fw_inter = fw.reshape(SK)
    return beid_f.astype(jnp.int32), valid, psc, pos_pl, fw_inter, NB, P


def _sc_dispatch(xf, psc, S, P, NW, n_ch, CHT):
    tok_per_w = S // NW
    mesh = plsc.VectorSubcoreMesh(core_axis_name="c", subcore_axis_name="s")

    @functools.partial(
        pl.kernel, mesh=mesh,
        out_type=jax.ShapeDtypeStruct((P, C), jnp.float32),
        scratch_types=[
            pltpu.VMEM((K * n_ch, CHT), jnp.int32),
            pltpu.VMEM((CHT, C), jnp.float32),
            pltpu.VMEM((CHT, C), jnp.float32),
            pltpu.SemaphoreType.DMA,
            pltpu.SemaphoreType.DMA,
            pltpu.SemaphoreType.DMA,
            pltpu.SemaphoreType.DMA,
        ],
    )
    def k(x_hbm, psc_hbm, out_hbm, idx_v, x0_v, x1_v, l0, l1, s0, s1):
        wid = lax.axis_index("s") * 2 + lax.axis_index("c")
        base = wid * tok_per_w

        pltpu.sync_copy(psc_hbm.at[wid], idx_v)

        xvs = (x0_v, x1_v); lsems = (l0, l1); ssems = (s0, s1)

        def start(c, slot):
            b = base + c * CHT
            pltpu.async_copy(x_hbm.at[pl.ds(b, CHT)], xvs[slot], lsems[slot])

        def finish(c, slot):
            b = base + c * CHT
            pltpu.make_async_copy(x_hbm.at[pl.ds(b, CHT)], xvs[slot],
                                  lsems[slot]).wait()
            pltpu.async_copy(xvs[slot], out_hbm.at[idx_v.at[K * c]],
                             ssems[slot])
            pltpu.async_copy(xvs[slot], out_hbm.at[idx_v.at[K * c + 1]],
                             ssems[slot])

        def sc_wait(c, slot):
            pltpu.make_async_copy(xvs[slot], out_hbm.at[idx_v.at[K * c]],
                                  ssems[slot]).wait()
            pltpu.make_async_copy(xvs[slot], out_hbm.at[idx_v.at[K * c + 1]],
                                  ssems[slot]).wait()

        start(0, 0)
        for c in range(n_ch):
            if c + 1 < n_ch:
                if c >= 1:
                    sc_wait(c - 1, (c + 1) % 2)
                start(c + 1, (c + 1) % 2)
            finish(c, c % 2)
        if n_ch >= 2:
            sc_wait(n_ch - 2, n_ch % 2)
        sc_wait(n_ch - 1, (n_ch - 1) % 2)

    return k(xf, psc)


def _grouped_body(eids_ref, valid_ref, xg_ref, gw_ref, uw_ref, dw_ref,
                  out_ref):
    b = pl.program_id(0)

    @pl.when(valid_ref[b] == 1)
    def _():
        xb = xg_ref[...]
        g = jnp.dot(xb, gw_ref[0], preferred_element_type=jnp.float32)
        u = jnp.dot(xb, uw_ref[0], preferred_element_type=jnp.float32)
        h = (g * jax.nn.sigmoid(g)) * u
        out_ref[...] = jnp.dot(h, dw_ref[0], preferred_element_type=jnp.float32)


def _grouped_mm(xg, gate_w, up_w, down_w, eids, valid, NB, P):
    grid_spec = pltpu.PrefetchScalarGridSpec(
        num_scalar_prefetch=2,
        grid=(NB,),
        in_specs=[
            pl.BlockSpec((BLK, C), lambda i, e, v: (i, 0)),
            pl.BlockSpec((1, C, H), lambda i, e, v: (e[i], 0, 0)),
            pl.BlockSpec((1, C, H), lambda i, e, v: (e[i], 0, 0)),
            pl.BlockSpec((1, H, C), lambda i, e, v: (e[i], 0, 0)),
        ],
        out_specs=pl.BlockSpec((BLK, C), lambda i, e, v: (i, 0)),
    )
    return pl.pallas_call(
        _grouped_body,
        grid_spec=grid_spec,
        out_shape=jax.ShapeDtypeStruct((P, C), jnp.float32),
    )(eids, valid, xg, gate_w, up_w, down_w)


def _sc_combine(shared, yg, pos_pl, fw_inter, S):
    info = plsc.get_sparse_core_info()
    NW = info.num_cores * info.num_subcores
    rows_per_w = S // NW
    CH = 4
    DEPTH = 4
    n_ch = rows_per_w // CH
    mesh = plsc.VectorSubcoreMesh(core_axis_name="c", subcore_axis_name="s")

    @functools.partial(
        pl.kernel, mesh=mesh,
        out_type=jax.ShapeDtypeStruct((S, C), jnp.float32),
        compiler_params=pltpu.CompilerParams(needs_layout_passes=False),
        scratch_types=[
            pltpu.VMEM((2, rows_per_w), jnp.int32),
            pltpu.VMEM((2 * rows_per_w,), jnp.float32),
            pltpu.VMEM((CH, C), jnp.float32),
            pltpu.VMEM((CH, C), jnp.float32),
            pltpu.VMEM((CH, C), jnp.float32),
            pltpu.VMEM((CH, C), jnp.float32),
            pltpu.VMEM((CH, C), jnp.float32),
            pltpu.VMEM((CH, C), jnp.float32),
            pltpu.VMEM((CH, C), jnp.float32),
            pltpu.VMEM((CH, C), jnp.float32),
            pltpu.VMEM((CH, C), jnp.float32),
            pltpu.VMEM((CH, C), jnp.float32),
            pltpu.VMEM((CH, C), jnp.float32),
            pltpu.VMEM((CH, C), jnp.float32),
            pltpu.SemaphoreType.DMA,
            pltpu.SemaphoreType.DMA,
            pltpu.SemaphoreType.DMA,
            pltpu.SemaphoreType.DMA,
            pltpu.SemaphoreType.DMA,
            pltpu.SemaphoreType.DMA,
            pltpu.SemaphoreType.DMA,
            pltpu.SemaphoreType.DMA,
            pltpu.SemaphoreType.DMA,
            pltpu.SemaphoreType.DMA,
            pltpu.SemaphoreType.DMA,
            pltpu.SemaphoreType.DMA,
        ],
    )
    def k(sh_hbm, yg_hbm, pp_hbm, fwi_hbm, out_hbm,
          iv_all, wv_all,
          a0, b0, s0, a1, b1, s1, a2, b2, s2, a3, b3, s3,
          ga0, gb0, sh0, ga1, gb1, sh1, ga2, gb2, sh2, ga3, gb3, sh3):
        wid = lax.axis_index("s") * info.num_cores + lax.axis_index("c")
        base = wid * rows_per_w

        pltpu.sync_copy(pp_hbm.at[wid], iv_all)
        pltpu.sync_copy(fwi_hbm.at[pl.ds(2 * base, 2 * rows_per_w)], wv_all)

        avs = (a0, a1, a2, a3); bvs = (b0, b1, b2, b3)
        svs = (s0, s1, s2, s3)
        gase = (ga0, ga1, ga2, ga3); gbse = (gb0, gb1, gb2, gb3)
        ssems = (sh0, sh1, sh2, sh3)

        def start(c, slot):
            b = base + c * CH
            i0 = iv_all.at[0, pl.ds(c * CH, CH)]
            i1 = iv_all.at[1, pl.ds(c * CH, CH)]
            pltpu.async_copy(yg_hbm.at[i0], avs[slot], gase[slot])
            pltpu.async_copy(yg_hbm.at[i1], bvs[slot], gbse[slot])
            pltpu.async_copy(sh_hbm.at[pl.ds(b, CH)], svs[slot], ssems[slot])

        def finish(c, slot):
            b = base + c * CH
            i0 = iv_all.at[0, pl.ds(c * CH, CH)]
            i1 = iv_all.at[1, pl.ds(c * CH, CH)]
            pltpu.make_async_copy(yg_hbm.at[i0], avs[slot], gase[slot]).wait()
            pltpu.make_async_copy(yg_hbm.at[i1], bvs[slot], gbse[slot]).wait()
            pltpu.make_async_copy(sh_hbm.at[pl.ds(b, CH)], svs[slot],
                                  ssems[slot]).wait()
            sv = svs[slot]; av = avs[slot]; bv = bvs[slot]

            def add_row(r, _2):
                q = 2 * (c * CH + r)
                w0 = plsc.load_gather(wv_all, [jnp.full((16,), q, jnp.int32)])
                w1 = plsc.load_gather(wv_all,
                                      [jnp.full((16,), q + 1, jnp.int32)])

                def add_grp(j, _3):
                    for u in range(16):
                        sl = pl.ds(j * 256 + u * 16, 16)
                        sv[r, sl] = (sv[r, sl] + w0 * av[r, sl]
                                     + w1 * bv[r, sl])
                    return _3
                lax.fori_loop(0, C // 256, add_grp, 0)
                return _2

            lax.fori_loop(0, CH, add_row, 0)
            pltpu.async_copy(sv, out_hbm.at[pl.ds(b, CH)], ssems[slot])

        def wb_wait(c, slot):
            b = base + c * CH
            pltpu.make_async_copy(svs[slot], out_hbm.at[pl.ds(b, CH)],
                                  ssems[slot]).wait()

        for p in range(min(DEPTH - 1, n_ch)):
            start(p, p % DEPTH)
        for c in range(n_ch):
            nxt = c + DEPTH - 1
            if nxt < n_ch:
                if c >= 1:
                    wb_wait(c - 1, nxt % DEPTH)
                start(nxt, nxt % DEPTH)
            finish(c, c % DEPTH)
        for t in range(max(0, n_ch - DEPTH), n_ch):
            wb_wait(t, t % DEPTH)

    return k(shared, yg, pos_pl, fw_inter)


def kernel(x, router_w, corr_bias, gate_w, up_w, down_w,
           shared_gate_w, shared_up_w, shared_down_w):
    Bx, Tx, Cx = x.shape
    S = Bx * Tx
    xf = x.reshape(S, Cx)

    NW = 32
    CHT = 16
    n_ch = S // NW // CHT

    topk, fw = _router(xf, router_w, corr_bias)
    (eids, valid, psc, pos_pl, fw_inter, NB, P) = _dispatch_metadata(
        topk, fw, S, NW, n_ch, CHT)

    xg = _sc_dispatch(xf, psc, S, P, NW, n_ch, CHT)
    shared = _shared_expert(xf, shared_gate_w, shared_up_w, shared_down_w)
    yg = _grouped_mm(xg, gate_w, up_w, down_w, eids, valid, NB, P)
    out = _sc_combine(shared, yg, pos_pl, fw_inter, S)
    return out.reshape(Bx, Tx, Cx)


# speedup vs baseline: 1.0123x; 1.0123x over previous
import functools

import jax
import jax.numpy as jnp
from jax import lax
from jax.experimental import pallas as pl
from jax.experimental.pallas import tpu as pltpu
from jax.experimental.pallas import tpu_sc as plsc

E = 16; G = 4; EPG = 4; K = 2; TOPK_GROUP = 2
C = 2048; H = 1024; H_SHARED = 1024

BLK = 256
NEG_INF = float("-inf")


def _num_blocks(S):
    SK = S * K
    return -(-(SK + E * (BLK - 1)) // BLK)


def _router_body(x_ref, rw_ref, rb_ref, topk_ref, fw_ref):
    xb = x_ref[...]
    logits = jax.lax.dot_general(xb, rw_ref[...], (((1,), (1,)), ((), ())),
                                 preferred_element_type=jnp.float32)
    scores = jax.nn.sigmoid(logits)
    sb = scores + rb_ref[...]

    TS = xb.shape[0]
    le = lax.broadcasted_iota(jnp.int32, (TS, E), 1)
    gl = le // EPG

    gcols = []
    for g_id in range(G):
        gm = jnp.max(jnp.where(gl == g_id, sb, NEG_INF), axis=1, keepdims=True)
        gcols.append(gm)
    gsc = jnp.concatenate(gcols, axis=1)

    gi = lax.broadcasted_iota(jnp.int32, (TS, G), 1)
    m1 = jnp.max(gsc, axis=1, keepdims=True)
    g1 = jnp.min(jnp.where(gsc == m1, gi, G), axis=1, keepdims=True)
    gsc2 = jnp.where(gi == g1, NEG_INF, gsc)
    m2 = jnp.max(gsc2, axis=1, keepdims=True)
    g2 = jnp.min(jnp.where(gsc2 == m2, gi, G), axis=1, keepdims=True)

    gmask = (gl == g1) | (gl == g2)
    msb = jnp.where(gmask, sb, NEG_INF)

    v1 = jnp.max(msb, axis=1, keepdims=True)
    e1 = jnp.min(jnp.where(msb == v1, le, E), axis=1, keepdims=True)
    msb2 = jnp.where(le == e1, NEG_INF, msb)
    v2 = jnp.max(msb2, axis=1, keepdims=True)
    e2 = jnp.min(jnp.where(msb2 == v2, le, E), axis=1, keepdims=True)

    w1 = jnp.sum(jnp.where(le == e1, scores, 0.0), axis=1, keepdims=True)
    w2 = jnp.sum(jnp.where(le == e2, scores, 0.0), axis=1, keepdims=True)
    norm = w1 + w2 + 1e-20

    topk_ref[...] = jnp.concatenate([e1, e2], axis=1)
    fw_ref[...] = jnp.concatenate([w1 / norm, w2 / norm], axis=1)


def _router(xf, router_w, corr_bias):
    S = xf.shape[0]
    TS = 512
    return pl.pallas_call(
        _router_body,
        grid=(S // TS,),
        in_specs=[
            pl.BlockSpec((TS, C), lambda i: (i, 0)),
            pl.BlockSpec((E, C), lambda i: (0, 0)),
            pl.BlockSpec((1, E), lambda i: (0, 0)),
        ],
        out_specs=[
            pl.BlockSpec((TS, K), lambda i: (i, 0)),
            pl.BlockSpec((TS, K), lambda i: (i, 0)),
        ],
        out_shape=[
            jax.ShapeDtypeStruct((S, K), jnp.int32),
            jax.ShapeDtypeStruct((S, K), jnp.float32),
        ],
    )(xf, router_w, corr_bias.reshape(1, E))


def _shared_body(x_ref, sg_ref, su_ref, sd_ref, out_ref):
    xb = x_ref[...]
    g = jax.lax.dot_general(xb, sg_ref[...], (((1,), (1,)), ((), ())),
                            preferred_element_type=jnp.float32)
    u = jax.lax.dot_general(xb, su_ref[...], (((1,), (1,)), ((), ())),
                            preferred_element_type=jnp.float32)
    h = (g * jax.nn.sigmoid(g)) * u
    out_ref[...] = jax.lax.dot_general(h, sd_ref[...], (((1,), (1,)), ((), ())),
                                       preferred_element_type=jnp.float32)


def _shared_expert(xf, sgw, suw, sdw):
    S = xf.shape[0]
    TS = 256
    return pl.pallas_call(
        _shared_body,
        grid=(S // TS,),
        in_specs=[
            pl.BlockSpec((TS, C), lambda i: (i, 0)),
            pl.BlockSpec((H_SHARED, C), lambda i: (0, 0)),
            pl.BlockSpec((H_SHARED, C), lambda i: (0, 0)),
            pl.BlockSpec((C, H_SHARED), lambda i: (0, 0)),
        ],
        out_specs=pl.BlockSpec((TS, C), lambda i: (i, 0)),
        out_shape=jax.ShapeDtypeStruct((S, C), jnp.float32),
    )(xf, sgw, suw, sdw)


def _dispatch_metadata(topk, fw, S, NW, n_ch, CHT):
    SK = S * K
    NB = _num_blocks(S)
    P = NB * BLK

    e_flat = topk.reshape(SK)
    oh = e_flat[:, None] == jnp.arange(E, dtype=jnp.int32)[None, :]
    ohi = oh.astype(jnp.int32)
    cum = jnp.cumsum(ohi, axis=0)
    counts = cum[-1]
    rank = jnp.sum(jnp.where(oh, cum - 1, 0), axis=1)

    padded = -(-counts // BLK) * BLK
    pad_end = jnp.cumsum(padded).astype(jnp.int32)
    pad_start = pad_end - padded
    ppos = pad_start[e_flat] + rank

    bstart = jnp.arange(NB, dtype=jnp.int32) * BLK
    beid = jnp.searchsorted(pad_end, bstart, side="right").astype(jnp.int32)
    beidc = jnp.minimum(beid, E - 1)
    valid = ((beid < E) &
             ((bstart - pad_start[beidc]) < counts[beidc])).astype(jnp.int32)
    beid_f = lax.associative_scan(jnp.maximum, jnp.where(valid == 1, beidc, 0))

    psc = (ppos.reshape(NW, n_ch, CHT, K)
               .transpose(0, 1, 3, 2)
               .reshape(NW, K * n_ch, CHT))
    pos_inter = ppos
    fw_inter = fw.reshape(SK)
    return beid_f.astype(jnp.int32), valid, psc, pos_inter, fw_inter, NB, P


def _sc_dispatch(xf, psc, S, P, NW, n_ch, CHT):
    tok_per_w = S // NW
    mesh = plsc.VectorSubcoreMesh(core_axis_name="c", subcore_axis_name="s")

    @functools.partial(
        pl.kernel, mesh=mesh,
        out_type=jax.ShapeDtypeStruct((P, C), jnp.float32),
        scratch_types=[
            pltpu.VMEM((K * n_ch, CHT), jnp.int32),
            pltpu.VMEM((CHT, C), jnp.float32),
            pltpu.VMEM((CHT, C), jnp.float32),
            pltpu.SemaphoreType.DMA,
            pltpu.SemaphoreType.DMA,
            pltpu.SemaphoreType.DMA,
            pltpu.SemaphoreType.DMA,
        ],
    )
    def k(x_hbm, psc_hbm, out_hbm, idx_v, x0_v, x1_v, l0, l1, s0, s1):
        wid = lax.axis_index("s") * 2 + lax.axis_index("c")
        base = wid * tok_per_w

        pltpu.sync_copy(psc_hbm.at[wid], idx_v)

        xvs = (x0_v, x1_v); lsems = (l0, l1); ssems = (s0, s1)

        def start(c, slot):
            b = base + c * CHT
            pltpu.async_copy(x_hbm.at[pl.ds(b, CHT)], xvs[slot], lsems[slot])

        def finish(c, slot):
            b = base + c * CHT
            pltpu.make_async_copy(x_hbm.at[pl.ds(b, CHT)], xvs[slot],
                                  lsems[slot]).wait()
            pltpu.async_copy(xvs[slot], out_hbm.at[idx_v.at[K * c]],
                             ssems[slot])
            pltpu.async_copy(xvs[slot], out_hbm.at[idx_v.at[K * c + 1]],
                             ssems[slot])

        def sc_wait(c, slot):
            pltpu.make_async_copy(xvs[slot], out_hbm.at[idx_v.at[K * c]],
                                  ssems[slot]).wait()
            pltpu.make_async_copy(xvs[slot], out_hbm.at[idx_v.at[K * c + 1]],
                                  ssems[slot]).wait()

        start(0, 0)
        for c in range(n_ch):
            if c + 1 < n_ch:
                if c >= 1:
                    sc_wait(c - 1, (c + 1) % 2)
                start(c + 1, (c + 1) % 2)
            finish(c, c % 2)
        if n_ch >= 2:
            sc_wait(n_ch - 2, n_ch % 2)
        sc_wait(n_ch - 1, (n_ch - 1) % 2)

    return k(xf, psc)


def _grouped_body(eids_ref, valid_ref, xg_ref, gw_ref, uw_ref, dw_ref,
                  out_ref):
    b = pl.program_id(0)

    @pl.when(valid_ref[b] == 1)
    def _():
        xb = xg_ref[...]
        g = jnp.dot(xb, gw_ref[0], preferred_element_type=jnp.float32)
        u = jnp.dot(xb, uw_ref[0], preferred_element_type=jnp.float32)
        h = (g * jax.nn.sigmoid(g)) * u
        out_ref[...] = jnp.dot(h, dw_ref[0], preferred_element_type=jnp.float32)


def _grouped_mm(xg, gate_w, up_w, down_w, eids, valid, NB, P):
    grid_spec = pltpu.PrefetchScalarGridSpec(
        num_scalar_prefetch=2,
        grid=(NB,),
        in_specs=[
            pl.BlockSpec((BLK, C), lambda i, e, v: (i, 0)),
            pl.BlockSpec((1, C, H), lambda i, e, v: (e[i], 0, 0)),
            pl.BlockSpec((1, C, H), lambda i, e, v: (e[i], 0, 0)),
            pl.BlockSpec((1, H, C), lambda i, e, v: (e[i], 0, 0)),
        ],
        out_specs=pl.BlockSpec((BLK, C), lambda i, e, v: (i, 0)),
    )
    return pl.pallas_call(
        _grouped_body,
        grid_spec=grid_spec,
        out_shape=jax.ShapeDtypeStruct((P, C), jnp.float32),
    )(eids, valid, xg, gate_w, up_w, down_w)


def _sc_combine(shared, yg, pos_inter, fw_inter, S):
    info = plsc.get_sparse_core_info()
    NW = info.num_cores * info.num_subcores
    rows_per_w = S // NW
    CH = 4
    DEPTH = 4
    n_ch = rows_per_w // CH
    mesh = plsc.VectorSubcoreMesh(core_axis_name="c", subcore_axis_name="s")

    @functools.partial(
        pl.kernel, mesh=mesh,
        out_type=jax.ShapeDtypeStruct((S, C), jnp.float32),
        compiler_params=pltpu.CompilerParams(needs_layout_passes=False),
        scratch_types=[
            pltpu.VMEM((2 * rows_per_w,), jnp.int32),
            pltpu.VMEM((2 * rows_per_w,), jnp.float32),
            pltpu.VMEM((2 * CH, C), jnp.float32),
            pltpu.VMEM((CH, C), jnp.float32),
            pltpu.VMEM((2 * CH, C), jnp.float32),
            pltpu.VMEM((CH, C), jnp.float32),
            pltpu.VMEM((2 * CH, C), jnp.float32),
            pltpu.VMEM((CH, C), jnp.float32),
            pltpu.VMEM((2 * CH, C), jnp.float32),
            pltpu.VMEM((CH, C), jnp.float32),
            pltpu.SemaphoreType.DMA,
            pltpu.SemaphoreType.DMA,
            pltpu.SemaphoreType.DMA,
            pltpu.SemaphoreType.DMA,
            pltpu.SemaphoreType.DMA,
            pltpu.SemaphoreType.DMA,
            pltpu.SemaphoreType.DMA,
            pltpu.SemaphoreType.DMA,
        ],
    )
    def k(sh_hbm, yg_hbm, pi_hbm, fwi_hbm, out_hbm,
          iv_all, wv_all, ra, sa, rb, sb_, rc, sc_, rd, sd_,
          ga, sha, gb, shb, gc, shc, gd, shd):
        wid = lax.axis_index("s") * info.num_cores + lax.axis_index("c")
        base = wid * rows_per_w

        pltpu.sync_copy(pi_hbm.at[pl.ds(2 * base, 2 * rows_per_w)], iv_all)
        pltpu.sync_copy(fwi_hbm.at[pl.ds(2 * base, 2 * rows_per_w)], wv_all)

        rvs = (ra, rb, rc, rd); svs = (sa, sb_, sc_, sd_)
        gsems = (ga, gb, gc, gd); ssems = (sha, shb, shc, shd)

        def start(c, slot):
            b = base + c * CH
            idx = iv_all.at[pl.ds(2 * c * CH, 2 * CH)]
            pltpu.async_copy(yg_hbm.at[idx], rvs[slot], gsems[slot])
            pltpu.async_copy(sh_hbm.at[pl.ds(b, CH)], svs[slot], ssems[slot])

        def finish(c, slot):
            b = base + c * CH
            idx = iv_all.at[pl.ds(2 * c * CH, 2 * CH)]
            pltpu.make_async_copy(yg_hbm.at[idx], rvs[slot],
                                  gsems[slot]).wait()
            pltpu.make_async_copy(sh_hbm.at[pl.ds(b, CH)], svs[slot],
                                  ssems[slot]).wait()
            sv = svs[slot]; rv = rvs[slot]

            def add_row(r, _2):
                q = 2 * (c * CH + r)
                w0 = plsc.load_gather(wv_all, [jnp.full((16,), q, jnp.int32)])
                w1 = plsc.load_gather(wv_all,
                                      [jnp.full((16,), q + 1, jnp.int32)])

                def add_grp(j, _3):
                    for u in range(16):
                        sl = pl.ds(j * 256 + u * 16, 16)
                        sv[r, sl] = (sv[r, sl] + w0 * rv[2 * r, sl]
                                     + w1 * rv[2 * r + 1, sl])
                    return _3
                lax.fori_loop(0, C // 256, add_grp, 0)
                return _2

            lax.fori_loop(0, CH, add_row, 0)
            pltpu.async_copy(sv, out_hbm.at[pl.ds(b, CH)], ssems[slot])

        def wb_wait(c, slot):
            b = base + c * CH
            pltpu.make_async_copy(svs[slot], out_hbm.at[pl.ds(b, CH)],
                                  ssems[slot]).wait()

        for p in range(min(DEPTH - 1, n_ch)):
            start(p, p % DEPTH)
        for c in range(n_ch):
            nxt = c + DEPTH - 1
            if nxt < n_ch:
                if c >= 1:
                    wb_wait(c - 1, nxt % DEPTH)
                start(nxt, nxt % DEPTH)
            finish(c, c % DEPTH)
        for t in range(max(0, n_ch - DEPTH), n_ch):
            wb_wait(t, t % DEPTH)

    return k(shared, yg, pos_inter, fw_inter)


def kernel(x, router_w, corr_bias, gate_w, up_w, down_w,
           shared_gate_w, shared_up_w, shared_down_w):
    Bx, Tx, Cx = x.shape
    S = Bx * Tx
    xf = x.reshape(S, Cx)

    NW = 32
    CHT = 8
    n_ch = S // NW // CHT

    topk, fw = _router(xf, router_w, corr_bias)
    (eids, valid, psc, pos_inter, fw_inter, NB, P) = _dispatch_metadata(
        topk, fw, S, NW, n_ch, CHT)

    xg = _sc_dispatch(xf, psc, S, P, NW, n_ch, CHT)
    shared = _shared_expert(xf, shared_gate_w, shared_up_w, shared_down_w)
    yg = _grouped_mm(xg, gate_w, up_w, down_w, eids, valid, NB, P)
    out = _sc_combine(shared, yg, pos_inter, fw_inter, S)
    return out.reshape(Bx, Tx, Cx)
